# baseline (speedup 1.0000x reference)
import functools

import jax
import jax.numpy as jnp
from jax import lax
from jax.experimental import pallas as pl
from jax.experimental.pallas import tpu as pltpu
from jax.experimental.pallas import tpu_sc as plsc

_NC = 2
_NS = 16
_NW = _NC * _NS
_L = 16


@functools.cache
def _build_table_widen(vocab, d, blk=1024):
    def body(in_ref, out_ref):
        i = lax.broadcasted_iota(jnp.int32, (d, d), 0)
        j = lax.broadcasted_iota(jnp.int32, (d, d), 1)
        eye = jnp.where(i == j, 1.0, 0.0).astype(jnp.float32)
        tr = lax.dot_general(in_ref[...], eye, (((0,), (0,)), ((), ())),
                             preferred_element_type=jnp.float32,
                             precision=lax.Precision.HIGHEST)
        out_ref[:, pl.ds(0, d)] = tr

    return pl.pallas_call(
        body,
        grid=((vocab + blk - 1) // blk,),
        in_specs=[pl.BlockSpec((d, blk), lambda j: (0, j))],
        out_specs=pl.BlockSpec((blk, 128), lambda j: (j, 0)),
        out_shape=jax.ShapeDtypeStruct((vocab, 128), jnp.float32),
    )


@functools.cache
def _build_idx_flatten(rows, batch):
    def body(in_ref, out_ref):
        for k in range(rows):
            out_ref[pl.ds(k * batch, batch)] = in_ref[k, :]

    return pl.pallas_call(
        body,
        out_shape=jax.ShapeDtypeStruct((rows * batch,), jnp.int32),
    )


@functools.cache
def _build_sc_scores(vocab, d, batch, ctx, neg):
    assert d % _L == 0
    bpw = batch // _NW
    unit = 16
    nunits = bpw // unit
    qn = d // _L

    mesh = plsc.VectorSubcoreMesh(core_axis_name="c", subcore_axis_name="s")

    @functools.partial(
        pl.kernel,
        out_type=(
            jax.ShapeDtypeStruct((batch,), jnp.float32),
            jax.ShapeDtypeStruct((batch // unit, neg, _L), jnp.float32),
        ),
        mesh=mesh,
        compiler_params=pltpu.CompilerParams(needs_layout_passes=False,
                                             use_tc_tiling_on_sc=False),
        scratch_types=[
            pltpu.VMEM((ctx, bpw), jnp.int32),
            pltpu.VMEM((neg, bpw), jnp.int32),
            pltpu.VMEM((bpw,), jnp.int32),
            pltpu.VMEM((ctx, unit, 128), jnp.float32),
            pltpu.VMEM((2, neg, unit, 128), jnp.float32),
            pltpu.VMEM((2, unit, 128), jnp.float32),
            pltpu.VMEM((unit * d,), jnp.float32),
            pltpu.VMEM((bpw,), jnp.float32),
            pltpu.VMEM((2, neg, _L), jnp.float32),
            pltpu.SemaphoreType.DMA,
            pltpu.SemaphoreType.DMA,
            pltpu.SemaphoreType.DMA,
            pltpu.SemaphoreType.DMA,
            pltpu.SemaphoreType.DMA,
        ],
    )
    def sc_scores(emb, ow, ctx_f, tgt_i, neg_f, pos_out, negt_out,
                  ctx_idx, neg_idx, tgt_idx, ctx_buf, neg_buf, tgt_buf,
                  avg_buf, pos_buf, negt_st, sem_s, sem_c, sem0, sem1,
                  sem_o):
        wid = lax.axis_index("s") * _NC + lax.axis_index("c")
        sems = (sem0, sem1)
        base = wid * bpw

        for k in range(ctx):
            pltpu.async_copy(ctx_f.at[pl.ds(k * batch + base, bpw)],
                             ctx_idx.at[k], sem_s)
        for n in range(neg):
            pltpu.async_copy(neg_f.at[pl.ds(n * batch + base, bpw)],
                             neg_idx.at[n], sem_s)
        pltpu.async_copy(tgt_i.at[pl.ds(base, bpw)], tgt_idx, sem_s)
        for k in range(ctx):
            pltpu.make_async_copy(ctx_f.at[pl.ds(k * batch + base, bpw)],
                                  ctx_idx.at[k], sem_s).wait()
        for n in range(neg):
            pltpu.make_async_copy(neg_f.at[pl.ds(n * batch + base, bpw)],
                                  neg_idx.at[n], sem_s).wait()
        pltpu.make_async_copy(tgt_i.at[pl.ds(base, bpw)], tgt_idx,
                              sem_s).wait()

        def fire_ctx(u):
            for k in range(ctx):
                pltpu.async_copy(emb.at[ctx_idx.at[k, pl.ds(u * unit, unit)]],
                                 ctx_buf.at[k], sem_c)

        def drain_ctx(u):
            for k in range(ctx):
                pltpu.make_async_copy(
                    emb.at[ctx_idx.at[k, pl.ds(u * unit, unit)]],
                    ctx_buf.at[k], sem_c).wait()

        def fire_nt(u, b):
            for n in range(neg):
                pltpu.async_copy(ow.at[neg_idx.at[n, pl.ds(u * unit, unit)]],
                                 neg_buf.at[b, n], sems[b])
            pltpu.async_copy(ow.at[tgt_idx.at[pl.ds(u * unit, unit)]],
                             tgt_buf.at[b], sems[b])

        def drain_nt(u, b):
            for n in range(neg):
                pltpu.make_async_copy(
                    ow.at[neg_idx.at[n, pl.ds(u * unit, unit)]],
                    neg_buf.at[b, n], sems[b]).wait()
            pltpu.make_async_copy(ow.at[tgt_idx.at[pl.ds(u * unit, unit)]],
                                  tgt_buf.at[b], sems[b]).wait()

        def wout_copy(u, b):
            return pltpu.make_async_copy(
                negt_st.at[b], negt_out.at[wid * nunits + u], sem_o)

        iota = lax.iota(jnp.int32, _L)

        def hsum(v):
            return plsc.cumsum(v)[_L - 1]

        def compute_avg(u):
            def row_body(r, carry):
                for q in range(qn):
                    acc = ctx_buf[0, r, pl.ds(q * _L, _L)]
                    for k in range(1, ctx):
                        acc = acc + ctx_buf[k, r, pl.ds(q * _L, _L)]
                    plsc.store_scatter(avg_buf,
                                       [r * d + q * _L + iota],
                                       acc * (1.0 / ctx))
                return carry

            lax.fori_loop(0, unit, row_body, 0)

        def compute_scores(u, b):
            def row_body(r, carry):
                pos_vec, nvecs = carry
                a = [avg_buf[pl.ds(r * d + q * _L, _L)] for q in range(qn)]
                e = a[0] * tgt_buf[b, r, pl.ds(0, _L)]
                for q in range(1, qn):
                    e = e + a[q] * tgt_buf[b, r, pl.ds(q * _L, _L)]
                pos_vec = jnp.where(iota == r, hsum(e), pos_vec)
                new_nvecs = []
                for n in range(neg):
                    e = a[0] * neg_buf[b, n, r, pl.ds(0, _L)]
                    for q in range(1, qn):
                        e = e + a[q] * neg_buf[b, n, r, pl.ds(q * _L, _L)]
                    new_nvecs.append(jnp.where(iota == r, hsum(e), nvecs[n]))
                return (pos_vec, tuple(new_nvecs))

            zero = jnp.zeros((_L,), jnp.float32)
            pos_vec, nvecs = lax.fori_loop(0, unit, row_body,
                                           (zero, (zero,) * neg))
            plsc.store_scatter(pos_buf, [u * unit + iota], pos_vec)
            for n in range(neg):
                negt_st[b, n, :] = nvecs[n]

        fire_ctx(0)
        fire_nt(0, 0)

        def pair_body(up, carry):
            for b in range(2):
                u = up * 2 + b

                @pl.when(u + 1 < nunits)
                def _fire_nt_next():
                    fire_nt(u + 1, 1 - b)

                drain_ctx(u)
                compute_avg(u)

                @pl.when(u + 1 < nunits)
                def _fire_ctx_next():
                    fire_ctx(u + 1)

                drain_nt(u, b)

                @pl.when(u >= 2)
                def _drain_wout():
                    wout_copy(u - 2, b).wait()

                compute_scores(u, b)
                wout_copy(u, b).start()
            return carry

        lax.fori_loop(0, nunits // 2, pair_body, 0)

        wout_copy(nunits - 2, 0).wait()
        wout_copy(nunits - 1, 1).wait()
        pltpu.sync_copy(pos_buf, pos_out.at[pl.ds(base, bpw)])

    return sc_scores


@functools.cache
def _build_tc_loss(batch, neg):
    def body(pos_ref, neg_ref, out_ref):
        p = pos_ref[...]
        s = neg_ref[...]
        sp_p = jnp.maximum(-p, 0.0) + jnp.log(1.0 + jnp.exp(-jnp.abs(p)))
        sp_n = jnp.maximum(s, 0.0) + jnp.log(1.0 + jnp.exp(-jnp.abs(s)))
        val = (jnp.sum(sp_p) * (1.0 / batch)
               + jnp.sum(sp_n) * (1.0 / (batch * neg)))
        out_ref[...] = val.reshape(1, 1)

    return pl.pallas_call(
        body,
        out_shape=jax.ShapeDtypeStruct((1, 1), jnp.float32),
    )


@jax.jit
def kernel(embeddings, output_weights, context, target, neg_samples):
    vocab, d = embeddings.shape
    batch, ctx = context.shape
    neg = neg_samples.shape[1]
    emb_w = _build_table_widen(vocab, d)(embeddings.T)
    ow_w = _build_table_widen(vocab, d)(output_weights.T)
    ctx_flat = _build_idx_flatten(ctx, batch)(context.T)
    neg_flat = _build_idx_flatten(neg, batch)(neg_samples.T)
    sc = _build_sc_scores(vocab, d, batch, ctx, neg)
    tc = _build_tc_loss(batch, neg)
    pos, negt = sc(emb_w, ow_w, ctx_flat, target, neg_flat)
    out = tc(pos.reshape(-1, 128), negt.reshape(-1, 128))
    return out[0, 0]


# speedup vs baseline: 1.1501x; 1.1501x over previous
import functools

import jax
import jax.numpy as jnp
from jax import lax
from jax.experimental import pallas as pl
from jax.experimental.pallas import tpu as pltpu
from jax.experimental.pallas import tpu_sc as plsc

_NC = 2
_NS = 16
_NW = _NC * _NS
_L = 16


@functools.cache
def _build_table_widen(vocab, d, blk=1024):
    def body(in_ref, out_ref):
        i = lax.broadcasted_iota(jnp.int32, (d, d), 0)
        j = lax.broadcasted_iota(jnp.int32, (d, d), 1)
        eye = jnp.where(i == j, 1.0, 0.0).astype(jnp.float32)
        tr = lax.dot_general(in_ref[...], eye, (((0,), (0,)), ((), ())),
                             preferred_element_type=jnp.float32,
                             precision=lax.Precision.DEFAULT)
        out_ref[:, pl.ds(0, d)] = tr

    return pl.pallas_call(
        body,
        grid=((vocab + blk - 1) // blk,),
        in_specs=[pl.BlockSpec((d, blk), lambda j: (0, j))],
        out_specs=pl.BlockSpec((blk, 128), lambda j: (j, 0)),
        out_shape=jax.ShapeDtypeStruct((vocab, 128), jnp.float32),
    )


@functools.cache
def _build_idx_flatten(rows, batch):
    def body(in_ref, out_ref):
        for k in range(rows):
            out_ref[pl.ds(k * batch, batch)] = in_ref[k, :]

    return pl.pallas_call(
        body,
        out_shape=jax.ShapeDtypeStruct((rows * batch,), jnp.int32),
    )


@functools.cache
def _build_sc_scores(vocab, d, batch, ctx, neg):
    assert d % _L == 0
    bpw = batch // _NW
    unit = 16
    nunits = bpw // unit
    qn = d // _L

    mesh = plsc.VectorSubcoreMesh(core_axis_name="c", subcore_axis_name="s")

    @functools.partial(
        pl.kernel,
        out_type=(
            jax.ShapeDtypeStruct((batch,), jnp.float32),
            jax.ShapeDtypeStruct((batch // unit, neg, _L), jnp.float32),
        ),
        mesh=mesh,
        compiler_params=pltpu.CompilerParams(needs_layout_passes=False,
                                             use_tc_tiling_on_sc=False),
        scratch_types=[
            pltpu.VMEM((ctx, bpw), jnp.int32),
            pltpu.VMEM((neg, bpw), jnp.int32),
            pltpu.VMEM((bpw,), jnp.int32),
            pltpu.VMEM((ctx, unit, 128), jnp.float32),
            pltpu.VMEM((2, neg, unit, 128), jnp.float32),
            pltpu.VMEM((2, unit, 128), jnp.float32),
            pltpu.VMEM((unit * d,), jnp.float32),
            pltpu.VMEM((bpw,), jnp.float32),
            pltpu.VMEM((2, neg, _L), jnp.float32),
            pltpu.SemaphoreType.DMA,
            pltpu.SemaphoreType.DMA,
            pltpu.SemaphoreType.DMA,
            pltpu.SemaphoreType.DMA,
            pltpu.SemaphoreType.DMA,
        ],
    )
    def sc_scores(emb, ow, ctx_f, tgt_i, neg_f, pos_out, negt_out,
                  ctx_idx, neg_idx, tgt_idx, ctx_buf, neg_buf, tgt_buf,
                  avg_buf, pos_buf, negt_st, sem_s, sem_c, sem0, sem1,
                  sem_o):
        wid = lax.axis_index("s") * _NC + lax.axis_index("c")
        sems = (sem0, sem1)
        base = wid * bpw

        for k in range(ctx):
            pltpu.async_copy(ctx_f.at[pl.ds(k * batch + base, bpw)],
                             ctx_idx.at[k], sem_s)
        for n in range(neg):
            pltpu.async_copy(neg_f.at[pl.ds(n * batch + base, bpw)],
                             neg_idx.at[n], sem_s)
        pltpu.async_copy(tgt_i.at[pl.ds(base, bpw)], tgt_idx, sem_s)
        for k in range(ctx):
            pltpu.make_async_copy(ctx_f.at[pl.ds(k * batch + base, bpw)],
                                  ctx_idx.at[k], sem_s).wait()
        for n in range(neg):
            pltpu.make_async_copy(neg_f.at[pl.ds(n * batch + base, bpw)],
                                  neg_idx.at[n], sem_s).wait()
        pltpu.make_async_copy(tgt_i.at[pl.ds(base, bpw)], tgt_idx,
                              sem_s).wait()

        def fire_ctx(u):
            for k in range(ctx):
                pltpu.async_copy(emb.at[ctx_idx.at[k, pl.ds(u * unit, unit)]],
                                 ctx_buf.at[k], sem_c)

        def drain_ctx(u):
            for k in range(ctx):
                pltpu.make_async_copy(
                    emb.at[ctx_idx.at[k, pl.ds(u * unit, unit)]],
                    ctx_buf.at[k], sem_c).wait()

        def fire_nt(u, b):
            for n in range(neg):
                pltpu.async_copy(ow.at[neg_idx.at[n, pl.ds(u * unit, unit)]],
                                 neg_buf.at[b, n], sems[b])
            pltpu.async_copy(ow.at[tgt_idx.at[pl.ds(u * unit, unit)]],
                             tgt_buf.at[b], sems[b])

        def drain_nt(u, b):
            for n in range(neg):
                pltpu.make_async_copy(
                    ow.at[neg_idx.at[n, pl.ds(u * unit, unit)]],
                    neg_buf.at[b, n], sems[b]).wait()
            pltpu.make_async_copy(ow.at[tgt_idx.at[pl.ds(u * unit, unit)]],
                                  tgt_buf.at[b], sems[b]).wait()

        def wout_copy(u, b):
            return pltpu.make_async_copy(
                negt_st.at[b], negt_out.at[wid * nunits + u], sem_o)

        iota = lax.iota(jnp.int32, _L)

        def hsum(v):
            return plsc.cumsum(v)[_L - 1]

        def compute_avg(u):
            def row_body(r, carry):
                for q in range(qn):
                    acc = ctx_buf[0, r, pl.ds(q * _L, _L)]
                    for k in range(1, ctx):
                        acc = acc + ctx_buf[k, r, pl.ds(q * _L, _L)]
                    plsc.store_scatter(avg_buf,
                                       [r * d + q * _L + iota],
                                       acc * (1.0 / ctx))
                return carry

            lax.fori_loop(0, unit, row_body, 0)

        def compute_scores(u, b):
            def row_body(r, carry):
                pos_vec, nvecs = carry
                a = [avg_buf[pl.ds(r * d + q * _L, _L)] for q in range(qn)]
                e = a[0] * tgt_buf[b, r, pl.ds(0, _L)]
                for q in range(1, qn):
                    e = e + a[q] * tgt_buf[b, r, pl.ds(q * _L, _L)]
                pos_vec = jnp.where(iota == r, hsum(e), pos_vec)
                new_nvecs = []
                for n in range(neg):
                    e = a[0] * neg_buf[b, n, r, pl.ds(0, _L)]
                    for q in range(1, qn):
                        e = e + a[q] * neg_buf[b, n, r, pl.ds(q * _L, _L)]
                    new_nvecs.append(jnp.where(iota == r, hsum(e), nvecs[n]))
                return (pos_vec, tuple(new_nvecs))

            zero = jnp.zeros((_L,), jnp.float32)
            pos_vec, nvecs = lax.fori_loop(0, unit, row_body,
                                           (zero, (zero,) * neg))
            plsc.store_scatter(pos_buf, [u * unit + iota], pos_vec)
            for n in range(neg):
                negt_st[b, n, :] = nvecs[n]

        fire_ctx(0)
        fire_nt(0, 0)

        def pair_body(up, carry):
            for b in range(2):
                u = up * 2 + b

                @pl.when(u + 1 < nunits)
                def _fire_nt_next():
                    fire_nt(u + 1, 1 - b)

                drain_ctx(u)
                compute_avg(u)

                @pl.when(u + 1 < nunits)
                def _fire_ctx_next():
                    fire_ctx(u + 1)

                drain_nt(u, b)

                @pl.when(u >= 2)
                def _drain_wout():
                    wout_copy(u - 2, b).wait()

                compute_scores(u, b)
                wout_copy(u, b).start()
            return carry

        lax.fori_loop(0, nunits // 2, pair_body, 0)

        wout_copy(nunits - 2, 0).wait()
        wout_copy(nunits - 1, 1).wait()
        pltpu.sync_copy(pos_buf, pos_out.at[pl.ds(base, bpw)])

    return sc_scores


@functools.cache
def _build_tc_loss(batch, neg):
    def body(pos_ref, neg_ref, out_ref):
        p = pos_ref[...]
        s = neg_ref[...]
        sp_p = jnp.maximum(-p, 0.0) + jnp.log(1.0 + jnp.exp(-jnp.abs(p)))
        sp_n = jnp.maximum(s, 0.0) + jnp.log(1.0 + jnp.exp(-jnp.abs(s)))
        val = (jnp.sum(sp_p) * (1.0 / batch)
               + jnp.sum(sp_n) * (1.0 / (batch * neg)))
        out_ref[...] = val.reshape(1, 1)

    return pl.pallas_call(
        body,
        out_shape=jax.ShapeDtypeStruct((1, 1), jnp.float32),
    )


@jax.jit
def kernel(embeddings, output_weights, context, target, neg_samples):
    vocab, d = embeddings.shape
    batch, ctx = context.shape
    neg = neg_samples.shape[1]
    emb_w = _build_table_widen(vocab, d)(embeddings.T)
    ow_w = _build_table_widen(vocab, d)(output_weights.T)
    ctx_flat = _build_idx_flatten(ctx, batch)(context.T)
    neg_flat = _build_idx_flatten(neg, batch)(neg_samples.T)
    sc = _build_sc_scores(vocab, d, batch, ctx, neg)
    tc = _build_tc_loss(batch, neg)
    pos, negt = sc(emb_w, ow_w, ctx_flat, target, neg_flat)
    out = tc(pos.reshape(-1, 128), negt.reshape(-1, 128))
    return out[0, 0]


# speedup vs baseline: 1.5934x; 1.3855x over previous
import functools

import jax
import jax.numpy as jnp
from jax import lax
from jax.experimental import pallas as pl
from jax.experimental.pallas import tpu as pltpu
from jax.experimental.pallas import tpu_sc as plsc

_NC = 2
_NS = 16
_NW = _NC * _NS
_L = 16


@functools.cache
def _build_table_widen(vocab, d, blk=1024):
    def body(in_ref, out_ref):
        i = lax.broadcasted_iota(jnp.int32, (d, d), 0)
        j = lax.broadcasted_iota(jnp.int32, (d, d), 1)
        eye = jnp.where(i == j, 1.0, 0.0).astype(jnp.float32)
        tr = lax.dot_general(in_ref[...], eye, (((0,), (0,)), ((), ())),
                             preferred_element_type=jnp.float32,
                             precision=lax.Precision.DEFAULT)
        out_ref[:, pl.ds(0, d)] = tr

    return pl.pallas_call(
        body,
        grid=((vocab + blk - 1) // blk,),
        in_specs=[pl.BlockSpec((d, blk), lambda j: (0, j))],
        out_specs=pl.BlockSpec((blk, 128), lambda j: (j, 0)),
        out_shape=jax.ShapeDtypeStruct((vocab, 128), jnp.float32),
    )


@functools.cache
def _build_idx_flatten(rows, batch):
    def body(in_ref, out_ref):
        for k in range(rows):
            out_ref[pl.ds(k * batch, batch)] = in_ref[k, :]

    return pl.pallas_call(
        body,
        out_shape=jax.ShapeDtypeStruct((rows * batch,), jnp.int32),
    )


@functools.cache
def _build_sc_scores(vocab, d, batch, ctx, neg):
    assert d % _L == 0
    bpw = batch // _NW
    unit = 16
    nunits = bpw // unit
    qn = d // _L

    mesh = plsc.VectorSubcoreMesh(core_axis_name="c", subcore_axis_name="s")

    @functools.partial(
        pl.kernel,
        out_type=(
            jax.ShapeDtypeStruct((batch,), jnp.float32),
            jax.ShapeDtypeStruct((batch // unit, neg, _L), jnp.float32),
        ),
        mesh=mesh,
        compiler_params=pltpu.CompilerParams(needs_layout_passes=False,
                                             use_tc_tiling_on_sc=False),
        scratch_types=[
            pltpu.VMEM((ctx, bpw), jnp.int32),
            pltpu.VMEM((neg, bpw), jnp.int32),
            pltpu.VMEM((bpw,), jnp.int32),
            pltpu.VMEM((ctx, unit, d), jnp.float32),
            pltpu.VMEM((2, neg, unit, d), jnp.float32),
            pltpu.VMEM((2, unit, d), jnp.float32),
            pltpu.VMEM((unit * d,), jnp.float32),
            pltpu.VMEM((bpw,), jnp.float32),
            pltpu.VMEM((2, neg, _L), jnp.float32),
            pltpu.SemaphoreType.DMA,
            pltpu.SemaphoreType.DMA,
            pltpu.SemaphoreType.DMA,
            pltpu.SemaphoreType.DMA,
            pltpu.SemaphoreType.DMA,
        ],
    )
    def sc_scores(emb, ow, ctx_f, tgt_i, neg_f, pos_out, negt_out,
                  ctx_idx, neg_idx, tgt_idx, ctx_buf, neg_buf, tgt_buf,
                  avg_buf, pos_buf, negt_st, sem_s, sem_c, sem0, sem1,
                  sem_o):
        wid = lax.axis_index("s") * _NC + lax.axis_index("c")
        sems = (sem0, sem1)
        base = wid * bpw

        for k in range(ctx):
            pltpu.async_copy(ctx_f.at[pl.ds(k * batch + base, bpw)],
                             ctx_idx.at[k], sem_s)
        for n in range(neg):
            pltpu.async_copy(neg_f.at[pl.ds(n * batch + base, bpw)],
                             neg_idx.at[n], sem_s)
        pltpu.async_copy(tgt_i.at[pl.ds(base, bpw)], tgt_idx, sem_s)
        for k in range(ctx):
            pltpu.make_async_copy(ctx_f.at[pl.ds(k * batch + base, bpw)],
                                  ctx_idx.at[k], sem_s).wait()
        for n in range(neg):
            pltpu.make_async_copy(neg_f.at[pl.ds(n * batch + base, bpw)],
                                  neg_idx.at[n], sem_s).wait()
        pltpu.make_async_copy(tgt_i.at[pl.ds(base, bpw)], tgt_idx,
                              sem_s).wait()

        def fire_ctx(u):
            for k in range(ctx):
                pltpu.async_copy(emb.at[ctx_idx.at[k, pl.ds(u * unit, unit)]],
                                 ctx_buf.at[k], sem_c)

        def drain_ctx(u):
            for k in range(ctx):
                pltpu.make_async_copy(
                    emb.at[ctx_idx.at[k, pl.ds(u * unit, unit)]],
                    ctx_buf.at[k], sem_c).wait()

        def fire_nt(u, b):
            for n in range(neg):
                pltpu.async_copy(ow.at[neg_idx.at[n, pl.ds(u * unit, unit)]],
                                 neg_buf.at[b, n], sems[b])
            pltpu.async_copy(ow.at[tgt_idx.at[pl.ds(u * unit, unit)]],
                             tgt_buf.at[b], sems[b])

        def drain_nt(u, b):
            for n in range(neg):
                pltpu.make_async_copy(
                    ow.at[neg_idx.at[n, pl.ds(u * unit, unit)]],
                    neg_buf.at[b, n], sems[b]).wait()
            pltpu.make_async_copy(ow.at[tgt_idx.at[pl.ds(u * unit, unit)]],
                                  tgt_buf.at[b], sems[b]).wait()

        def wout_copy(u, b):
            return pltpu.make_async_copy(
                negt_st.at[b], negt_out.at[wid * nunits + u], sem_o)

        iota = lax.iota(jnp.int32, _L)

        def hsum(v):
            return plsc.cumsum(v)[_L - 1]

        def compute_avg(u):
            def row_body(r, carry):
                for q in range(qn):
                    acc = ctx_buf[0, r, pl.ds(q * _L, _L)]
                    for k in range(1, ctx):
                        acc = acc + ctx_buf[k, r, pl.ds(q * _L, _L)]
                    plsc.store_scatter(avg_buf,
                                       [r * d + q * _L + iota],
                                       acc * (1.0 / ctx))
                return carry

            lax.fori_loop(0, unit, row_body, 0)

        def compute_scores(u, b):
            def row_body(r, carry):
                pos_vec, nvecs = carry
                a = [avg_buf[pl.ds(r * d + q * _L, _L)] for q in range(qn)]
                e = a[0] * tgt_buf[b, r, pl.ds(0, _L)]
                for q in range(1, qn):
                    e = e + a[q] * tgt_buf[b, r, pl.ds(q * _L, _L)]
                pos_vec = jnp.where(iota == r, hsum(e), pos_vec)
                new_nvecs = []
                for n in range(neg):
                    e = a[0] * neg_buf[b, n, r, pl.ds(0, _L)]
                    for q in range(1, qn):
                        e = e + a[q] * neg_buf[b, n, r, pl.ds(q * _L, _L)]
                    new_nvecs.append(jnp.where(iota == r, hsum(e), nvecs[n]))
                return (pos_vec, tuple(new_nvecs))

            zero = jnp.zeros((_L,), jnp.float32)
            pos_vec, nvecs = lax.fori_loop(0, unit, row_body,
                                           (zero, (zero,) * neg))
            plsc.store_scatter(pos_buf, [u * unit + iota], pos_vec)
            for n in range(neg):
                negt_st[b, n, :] = nvecs[n]

        fire_ctx(0)
        fire_nt(0, 0)

        def pair_body(up, carry):
            for b in range(2):
                u = up * 2 + b

                @pl.when(u + 1 < nunits)
                def _fire_nt_next():
                    fire_nt(u + 1, 1 - b)

                drain_ctx(u)
                compute_avg(u)

                @pl.when(u + 1 < nunits)
                def _fire_ctx_next():
                    fire_ctx(u + 1)

                drain_nt(u, b)

                @pl.when(u >= 2)
                def _drain_wout():
                    wout_copy(u - 2, b).wait()

                compute_scores(u, b)
                wout_copy(u, b).start()
            return carry

        lax.fori_loop(0, nunits // 2, pair_body, 0)

        wout_copy(nunits - 2, 0).wait()
        wout_copy(nunits - 1, 1).wait()
        pltpu.sync_copy(pos_buf, pos_out.at[pl.ds(base, bpw)])

    return sc_scores


@functools.cache
def _build_tc_loss(batch, neg):
    def body(pos_ref, neg_ref, out_ref):
        p = pos_ref[...]
        s = neg_ref[...]
        sp_p = jnp.maximum(-p, 0.0) + jnp.log(1.0 + jnp.exp(-jnp.abs(p)))
        sp_n = jnp.maximum(s, 0.0) + jnp.log(1.0 + jnp.exp(-jnp.abs(s)))
        val = (jnp.sum(sp_p) * (1.0 / batch)
               + jnp.sum(sp_n) * (1.0 / (batch * neg)))
        out_ref[...] = val.reshape(1, 1)

    return pl.pallas_call(
        body,
        out_shape=jax.ShapeDtypeStruct((1, 1), jnp.float32),
    )


@jax.jit
def kernel(embeddings, output_weights, context, target, neg_samples):
    vocab, d = embeddings.shape
    batch, ctx = context.shape
    neg = neg_samples.shape[1]
    ctx_flat = _build_idx_flatten(ctx, batch)(context.T)
    neg_flat = _build_idx_flatten(neg, batch)(neg_samples.T)
    sc = _build_sc_scores(vocab, d, batch, ctx, neg)
    tc = _build_tc_loss(batch, neg)
    pos, negt = sc(embeddings, output_weights, ctx_flat, target, neg_flat)
    out = tc(pos.reshape(-1, 128), negt.reshape(-1, 128))
    return out[0, 0]


# speedup vs baseline: 1.6118x; 1.0115x over previous
import functools

import jax
import jax.numpy as jnp
from jax import lax
from jax.experimental import pallas as pl
from jax.experimental.pallas import tpu as pltpu
from jax.experimental.pallas import tpu_sc as plsc

_NC = 2
_NS = 16
_NW = _NC * _NS
_L = 16


@functools.cache
def _build_table_widen(vocab, d, blk=1024):
    def body(in_ref, out_ref):
        i = lax.broadcasted_iota(jnp.int32, (d, d), 0)
        j = lax.broadcasted_iota(jnp.int32, (d, d), 1)
        eye = jnp.where(i == j, 1.0, 0.0).astype(jnp.float32)
        tr = lax.dot_general(in_ref[...], eye, (((0,), (0,)), ((), ())),
                             preferred_element_type=jnp.float32,
                             precision=lax.Precision.DEFAULT)
        out_ref[:, pl.ds(0, d)] = tr

    return pl.pallas_call(
        body,
        grid=((vocab + blk - 1) // blk,),
        in_specs=[pl.BlockSpec((d, blk), lambda j: (0, j))],
        out_specs=pl.BlockSpec((blk, 128), lambda j: (j, 0)),
        out_shape=jax.ShapeDtypeStruct((vocab, 128), jnp.float32),
    )


@functools.cache
def _build_idx_flatten(rows, batch):
    def body(in_ref, out_ref):
        for k in range(rows):
            out_ref[pl.ds(k * batch, batch)] = in_ref[k, :]

    return pl.pallas_call(
        body,
        out_shape=jax.ShapeDtypeStruct((rows * batch,), jnp.int32),
    )


@functools.cache
def _build_sc_scores(vocab, d, batch, ctx, neg):
    assert d % _L == 0
    bpw = batch // _NW
    unit = 16
    nunits = bpw // unit
    qn = d // _L

    mesh = plsc.VectorSubcoreMesh(core_axis_name="c", subcore_axis_name="s")

    @functools.partial(
        pl.kernel,
        out_type=(
            jax.ShapeDtypeStruct((batch,), jnp.float32),
            jax.ShapeDtypeStruct((batch * neg,), jnp.float32),
        ),
        mesh=mesh,
        compiler_params=pltpu.CompilerParams(needs_layout_passes=False,
                                             use_tc_tiling_on_sc=False),
        scratch_types=[
            pltpu.VMEM((ctx, bpw), jnp.int32),
            pltpu.VMEM((neg, bpw), jnp.int32),
            pltpu.VMEM((bpw,), jnp.int32),
            pltpu.VMEM((2, ctx, unit, d), jnp.float32),
            pltpu.VMEM((2, neg, unit, d), jnp.float32),
            pltpu.VMEM((2, unit, d), jnp.float32),
            pltpu.VMEM((bpw,), jnp.float32),
            pltpu.VMEM((nunits * neg * _L,), jnp.float32),
            pltpu.SemaphoreType.DMA,
            pltpu.SemaphoreType.DMA,
            pltpu.SemaphoreType.DMA,
        ],
    )
    def sc_scores(emb, ow, ctx_f, tgt_i, neg_f, pos_out, negt_out,
                  ctx_idx, neg_idx, tgt_idx, ctx_buf, neg_buf, tgt_buf,
                  pos_buf, negt_buf, sem_s, sem0, sem1):
        wid = lax.axis_index("s") * _NC + lax.axis_index("c")
        sems = (sem0, sem1)
        base = wid * bpw

        for k in range(ctx):
            pltpu.async_copy(ctx_f.at[pl.ds(k * batch + base, bpw)],
                             ctx_idx.at[k], sem_s)
        for n in range(neg):
            pltpu.async_copy(neg_f.at[pl.ds(n * batch + base, bpw)],
                             neg_idx.at[n], sem_s)
        pltpu.async_copy(tgt_i.at[pl.ds(base, bpw)], tgt_idx, sem_s)
        for k in range(ctx):
            pltpu.make_async_copy(ctx_f.at[pl.ds(k * batch + base, bpw)],
                                  ctx_idx.at[k], sem_s).wait()
        for n in range(neg):
            pltpu.make_async_copy(neg_f.at[pl.ds(n * batch + base, bpw)],
                                  neg_idx.at[n], sem_s).wait()
        pltpu.make_async_copy(tgt_i.at[pl.ds(base, bpw)], tgt_idx,
                              sem_s).wait()

        def fire(u, b):
            for k in range(ctx):
                pltpu.async_copy(emb.at[ctx_idx.at[k, pl.ds(u * unit, unit)]],
                                 ctx_buf.at[b, k], sems[b])
            for n in range(neg):
                pltpu.async_copy(ow.at[neg_idx.at[n, pl.ds(u * unit, unit)]],
                                 neg_buf.at[b, n], sems[b])
            pltpu.async_copy(ow.at[tgt_idx.at[pl.ds(u * unit, unit)]],
                             tgt_buf.at[b], sems[b])

        def drain(u, b):
            for k in range(ctx):
                pltpu.make_async_copy(
                    emb.at[ctx_idx.at[k, pl.ds(u * unit, unit)]],
                    ctx_buf.at[b, k], sems[b]).wait()
            for n in range(neg):
                pltpu.make_async_copy(
                    ow.at[neg_idx.at[n, pl.ds(u * unit, unit)]],
                    neg_buf.at[b, n], sems[b]).wait()
            pltpu.make_async_copy(ow.at[tgt_idx.at[pl.ds(u * unit, unit)]],
                                  tgt_buf.at[b], sems[b]).wait()

        iota = lax.iota(jnp.int32, _L)

        def hsum(v):
            return plsc.cumsum(v)[_L - 1]

        def compute(u, b):
            def row_body(r, carry):
                pos_vec, nvecs = carry
                a = []
                for q in range(qn):
                    acc = ctx_buf[b, 0, r, pl.ds(q * _L, _L)]
                    for k in range(1, ctx):
                        acc = acc + ctx_buf[b, k, r, pl.ds(q * _L, _L)]
                    a.append(acc * (1.0 / ctx))
                e = a[0] * tgt_buf[b, r, pl.ds(0, _L)]
                for q in range(1, qn):
                    e = e + a[q] * tgt_buf[b, r, pl.ds(q * _L, _L)]
                pos_vec = jnp.where(iota == r, hsum(e), pos_vec)
                new_nvecs = []
                for n in range(neg):
                    e = a[0] * neg_buf[b, n, r, pl.ds(0, _L)]
                    for q in range(1, qn):
                        e = e + a[q] * neg_buf[b, n, r, pl.ds(q * _L, _L)]
                    new_nvecs.append(jnp.where(iota == r, hsum(e), nvecs[n]))
                return (pos_vec, tuple(new_nvecs))

            zero = jnp.zeros((_L,), jnp.float32)
            pos_vec, nvecs = lax.fori_loop(0, unit, row_body,
                                           (zero, (zero,) * neg))
            plsc.store_scatter(pos_buf, [u * unit + iota], pos_vec)
            for n in range(neg):
                plsc.store_scatter(negt_buf, [(u * neg + n) * _L + iota],
                                   nvecs[n])

        fire(0, 0)

        def pair_body(up, carry):
            for b in range(2):
                u = up * 2 + b

                @pl.when(u + 1 < nunits)
                def _fire_next():
                    fire(u + 1, 1 - b)

                drain(u, b)
                compute(u, b)
            return carry

        lax.fori_loop(0, nunits // 2, pair_body, 0)

        pltpu.sync_copy(negt_buf,
                        negt_out.at[pl.ds(wid * nunits * neg * _L,
                                          nunits * neg * _L)])
        pltpu.sync_copy(pos_buf, pos_out.at[pl.ds(base, bpw)])

    return sc_scores


@functools.cache
def _build_tc_loss(batch, neg):
    def body(pos_ref, neg_ref, out_ref):
        p = pos_ref[...]
        s = neg_ref[...]
        sp_p = jnp.maximum(-p, 0.0) + jnp.log(1.0 + jnp.exp(-jnp.abs(p)))
        sp_n = jnp.maximum(s, 0.0) + jnp.log(1.0 + jnp.exp(-jnp.abs(s)))
        val = (jnp.sum(sp_p) * (1.0 / batch)
               + jnp.sum(sp_n) * (1.0 / (batch * neg)))
        out_ref[...] = val.reshape(1, 1)

    return pl.pallas_call(
        body,
        out_shape=jax.ShapeDtypeStruct((1, 1), jnp.float32),
    )


@jax.jit
def kernel(embeddings, output_weights, context, target, neg_samples):
    vocab, d = embeddings.shape
    batch, ctx = context.shape
    neg = neg_samples.shape[1]
    ctx_flat = _build_idx_flatten(ctx, batch)(context.T)
    neg_flat = _build_idx_flatten(neg, batch)(neg_samples.T)
    sc = _build_sc_scores(vocab, d, batch, ctx, neg)
    tc = _build_tc_loss(batch, neg)
    pos, negt = sc(embeddings, output_weights, ctx_flat, target, neg_flat)
    out = tc(pos.reshape(-1, 128), negt.reshape(-1, 128))
    return out[0, 0]


# speedup vs baseline: 1.6137x; 1.0012x over previous
import functools

import jax
import jax.numpy as jnp
from jax import lax
from jax.experimental import pallas as pl
from jax.experimental.pallas import tpu as pltpu
from jax.experimental.pallas import tpu_sc as plsc

_NC = 2
_NS = 16
_NW = _NC * _NS
_L = 16


@functools.cache
def _build_idx_flatten(rows, batch):
    def body(in_ref, out_ref):
        for k in range(rows):
            out_ref[pl.ds(k * batch, batch)] = in_ref[k, :]

    return pl.pallas_call(
        body,
        out_shape=jax.ShapeDtypeStruct((rows * batch,), jnp.int32),
    )


@functools.cache
def _build_sc_scores(vocab, d, batch, ctx, neg):
    assert d % _L == 0
    bpw = batch // _NW
    unit = 16
    nunits = bpw // unit
    qn = d // _L

    mesh = plsc.VectorSubcoreMesh(core_axis_name="c", subcore_axis_name="s")

    @functools.partial(
        pl.kernel,
        out_type=(
            jax.ShapeDtypeStruct((batch,), jnp.float32),
            jax.ShapeDtypeStruct((batch * neg,), jnp.float32),
        ),
        mesh=mesh,
        compiler_params=pltpu.CompilerParams(needs_layout_passes=False,
                                             use_tc_tiling_on_sc=False),
        scratch_types=[
            pltpu.VMEM((ctx, bpw), jnp.int32),
            pltpu.VMEM((neg, bpw), jnp.int32),
            pltpu.VMEM((bpw,), jnp.int32),
            pltpu.VMEM((2, ctx, unit, d), jnp.float32),
            pltpu.VMEM((2, neg, unit, d), jnp.float32),
            pltpu.VMEM((2, unit, d), jnp.float32),
            pltpu.VMEM((bpw,), jnp.float32),
            pltpu.VMEM((nunits * neg * _L,), jnp.float32),
            pltpu.SemaphoreType.DMA,
            pltpu.SemaphoreType.DMA,
            pltpu.SemaphoreType.DMA,
        ],
    )
    def sc_scores(emb, ow, ctx_f, tgt_i, neg_f, pos_out, negt_out,
                  ctx_idx, neg_idx, tgt_idx, ctx_buf, neg_buf, tgt_buf,
                  pos_buf, negt_buf, sem_s, sem0, sem1):
        wid = lax.axis_index("s") * _NC + lax.axis_index("c")
        sems = (sem0, sem1)
        base = wid * bpw

        for k in range(ctx):
            pltpu.async_copy(ctx_f.at[pl.ds(k * batch + base, bpw)],
                             ctx_idx.at[k], sem_s)
        for n in range(neg):
            pltpu.async_copy(neg_f.at[pl.ds(n * batch + base, bpw)],
                             neg_idx.at[n], sem_s)
        pltpu.async_copy(tgt_i.at[pl.ds(base, bpw)], tgt_idx, sem_s)
        for k in range(ctx):
            pltpu.make_async_copy(ctx_f.at[pl.ds(k * batch + base, bpw)],
                                  ctx_idx.at[k], sem_s).wait()
        for n in range(neg):
            pltpu.make_async_copy(neg_f.at[pl.ds(n * batch + base, bpw)],
                                  neg_idx.at[n], sem_s).wait()
        pltpu.make_async_copy(tgt_i.at[pl.ds(base, bpw)], tgt_idx,
                              sem_s).wait()

        def fire(u, b):
            for k in range(ctx):
                pltpu.async_copy(emb.at[ctx_idx.at[k, pl.ds(u * unit, unit)]],
                                 ctx_buf.at[b, k], sems[b])
            for n in range(neg):
                pltpu.async_copy(ow.at[neg_idx.at[n, pl.ds(u * unit, unit)]],
                                 neg_buf.at[b, n], sems[b])
            pltpu.async_copy(ow.at[tgt_idx.at[pl.ds(u * unit, unit)]],
                             tgt_buf.at[b], sems[b])

        def drain(u, b):
            for k in range(ctx):
                pltpu.make_async_copy(
                    emb.at[ctx_idx.at[k, pl.ds(u * unit, unit)]],
                    ctx_buf.at[b, k], sems[b]).wait()
            for n in range(neg):
                pltpu.make_async_copy(
                    ow.at[neg_idx.at[n, pl.ds(u * unit, unit)]],
                    neg_buf.at[b, n], sems[b]).wait()
            pltpu.make_async_copy(ow.at[tgt_idx.at[pl.ds(u * unit, unit)]],
                                  tgt_buf.at[b], sems[b]).wait()

        iota = lax.iota(jnp.int32, _L)

        def hsum(v):
            return plsc.cumsum(v)[_L - 1]

        def compute(u, b):
            def row_body(r, carry):
                pos_vec, nvecs = carry
                a = []
                for q in range(qn):
                    acc = ctx_buf[b, 0, r, pl.ds(q * _L, _L)]
                    for k in range(1, ctx):
                        acc = acc + ctx_buf[b, k, r, pl.ds(q * _L, _L)]
                    a.append(acc * (1.0 / ctx))
                e = a[0] * tgt_buf[b, r, pl.ds(0, _L)]
                for q in range(1, qn):
                    e = e + a[q] * tgt_buf[b, r, pl.ds(q * _L, _L)]
                pos_vec = jnp.where(iota == r, hsum(e), pos_vec)
                new_nvecs = []
                for n in range(neg):
                    e = a[0] * neg_buf[b, n, r, pl.ds(0, _L)]
                    for q in range(1, qn):
                        e = e + a[q] * neg_buf[b, n, r, pl.ds(q * _L, _L)]
                    new_nvecs.append(jnp.where(iota == r, hsum(e), nvecs[n]))
                return (pos_vec, tuple(new_nvecs))

            zero = jnp.zeros((_L,), jnp.float32)
            pos_vec, nvecs = lax.fori_loop(0, unit, row_body,
                                           (zero, (zero,) * neg))
            plsc.store_scatter(pos_buf, [u * unit + iota], pos_vec)
            for n in range(neg):
                plsc.store_scatter(negt_buf, [(u * neg + n) * _L + iota],
                                   nvecs[n])

        fire(0, 0)

        def pair_body(up, carry):
            for b in range(2):
                u = up * 2 + b

                @pl.when(u + 1 < nunits)
                def _fire_next():
                    fire(u + 1, 1 - b)

                drain(u, b)
                compute(u, b)
            return carry

        lax.fori_loop(0, nunits // 2, pair_body, 0)

        pltpu.sync_copy(negt_buf,
                        negt_out.at[pl.ds(wid * nunits * neg * _L,
                                          nunits * neg * _L)])
        pltpu.sync_copy(pos_buf, pos_out.at[pl.ds(base, bpw)])

    return sc_scores


@functools.cache
def _build_tc_loss(batch, neg):
    def body(pos_ref, neg_ref, out_ref):
        p = pos_ref[...]
        s = neg_ref[...]
        sp_p = jnp.maximum(-p, 0.0) + jnp.log(1.0 + jnp.exp(-jnp.abs(p)))
        sp_n = jnp.maximum(s, 0.0) + jnp.log(1.0 + jnp.exp(-jnp.abs(s)))
        val = (jnp.sum(sp_p) * (1.0 / batch)
               + jnp.sum(sp_n) * (1.0 / (batch * neg)))
        out_ref[...] = val.reshape(1, 1)

    return pl.pallas_call(
        body,
        out_shape=jax.ShapeDtypeStruct((1, 1), jnp.float32),
    )


@jax.jit
def kernel(embeddings, output_weights, context, target, neg_samples):
    vocab, d = embeddings.shape
    batch, ctx = context.shape
    neg = neg_samples.shape[1]
    ctx_flat = _build_idx_flatten(ctx, batch)(context.T)
    neg_flat = _build_idx_flatten(neg, batch)(neg_samples.T)
    sc = _build_sc_scores(vocab, d, batch, ctx, neg)
    tc = _build_tc_loss(batch, neg)
    pos, negt = sc(embeddings, output_weights, ctx_flat, target, neg_flat)
    out = tc(pos.reshape(-1, 128), negt.reshape(-1, 128))
    return out[0, 0]
